# baseline (speedup 1.0000x reference)
import functools

import jax
import jax.numpy as jnp
from jax import lax
from jax.experimental import pallas as pl
from jax.experimental.pallas import tpu as pltpu
from jax.experimental.pallas import tpu_sc as plsc

N = 10000
E = 160000
D = 256

NC = 2
NS = 16
L = 16

NP = 10240
EP = 163840
PADNODE = 10200

HALF = NP // 2

MW = 272
CS = 8
GE = 1024

GA = 64
NGA = (EP // 32) // GA


def _silu(x):
    return x * jax.nn.sigmoid(x)


def _bdot(a, b):
    return jnp.dot(a.astype(jnp.bfloat16), b.astype(jnp.bfloat16),
                   preferred_element_type=jnp.float32)


def _prep_body(h_ref, hi_ref, w1a_ref, w1b_ref, be1_ref, hin_ref, ad_ref, as_ref):
    h_in = h_ref[...] + hi_ref[...]
    hin_ref[...] = h_in
    ad_ref[...] = _bdot(h_in, w1a_ref[...]) + be1_ref[...]
    as_ref[...] = _bdot(h_in, w1b_ref[...])


def _prep_call(hp, hip, w1a, w1b, be1, interpret=False):
    nb = NP // 256
    return pl.pallas_call(
        _prep_body,
        grid=(nb,),
        in_specs=[
            pl.BlockSpec((256, D), lambda i: (i, 0)),
            pl.BlockSpec((256, D), lambda i: (i, 0)),
            pl.BlockSpec((D, D), lambda i: (0, 0)),
            pl.BlockSpec((D, D), lambda i: (0, 0)),
            pl.BlockSpec((1, D), lambda i: (0, 0)),
        ],
        out_specs=[
            pl.BlockSpec((256, D), lambda i: (i, 0)),
            pl.BlockSpec((256, D), lambda i: (i, 0)),
            pl.BlockSpec((256, D), lambda i: (i, 0)),
        ],
        out_shape=[
            jax.ShapeDtypeStruct((NP, D), jnp.float32),
            jax.ShapeDtypeStruct((NP, D), jnp.float32),
            jax.ShapeDtypeStruct((NP, D), jnp.float32),
        ],
        interpret=interpret,
    )(hp, hip, w1a, w1b, be1)


def _edge_body(pre_ref, rel_ref, wrow_ref, we2_ref, be2_ref, wx1_ref, bx1_ref,
               wx2_ref, bx2_ref, mcat_ref):
    rel = rel_ref[...]
    d2 = jnp.sum(rel * rel, axis=1, keepdims=True)
    x = pre_ref[...] + d2 * wrow_ref[...]
    m1 = _silu(x)
    m2 = _silu(_bdot(m1, we2_ref[...]) + be2_ref[...])
    t = _silu(_bdot(m2, wx1_ref[...]) + bx1_ref[...])
    xw = jnp.sum(t * wx2_ref[...], axis=1, keepdims=True) + bx2_ref[...]
    nrows = rel.shape[0]
    mcat_ref[...] = jnp.concatenate(
        [m2, rel * xw, jnp.ones((nrows, 1), jnp.float32),
         jnp.zeros((nrows, MW - D - 5), jnp.float32)], axis=1)


def _edge_call(pre, rel4, wrow, we2, be2, wx1, bx1, wx2r, bx2, interpret=False):
    eb = 1024
    nb = EP // eb
    return pl.pallas_call(
        _edge_body,
        grid=(nb,),
        in_specs=[
            pl.BlockSpec((eb, D), lambda i: (i, 0)),
            pl.BlockSpec((eb, 4), lambda i: (i, 0)),
            pl.BlockSpec((1, D), lambda i: (0, 0)),
            pl.BlockSpec((D, D), lambda i: (0, 0)),
            pl.BlockSpec((1, D), lambda i: (0, 0)),
            pl.BlockSpec((D, D), lambda i: (0, 0)),
            pl.BlockSpec((1, D), lambda i: (0, 0)),
            pl.BlockSpec((1, D), lambda i: (0, 0)),
            pl.BlockSpec((1, 1), lambda i: (0, 0)),
        ],
        out_specs=[
            pl.BlockSpec((eb, MW), lambda i: (i, 0)),
        ],
        out_shape=[
            jax.ShapeDtypeStruct((EP, MW), jnp.float32),
        ],
        interpret=interpret,
    )(pre, rel4, wrow, we2, be2, wx1, bx1, wx2r, bx2)


def _node_body(hin_ref, agg_ref, cmp_ref, pos_ref, wh1a_ref, wh1b_ref,
               bh1_ref, wh2_ref, bh2_ref, hout_ref, pout_ref):
    h_in = hin_ref[...]
    z = (_bdot(h_in, wh1a_ref[...])
         + _bdot(agg_ref[...], wh1b_ref[...])
         + bh1_ref[...])
    upd = _bdot(_silu(z), wh2_ref[...]) + bh2_ref[...]
    hout_ref[...] = h_in + upd
    cm = jnp.sum(cmp_ref[0], axis=0)
    cnt = jnp.maximum(cm[:, 4:5], 1.0)
    pout_ref[...] = pos_ref[...] + cm[:, 0:4] / cnt


def _node_call(hinp, agg, cmparts, posp, wh1a, wh1b, bh1, wh2, bh2, interpret=False):
    nb = NP // 256
    blk_per_half = HALF // 256
    return pl.pallas_call(
        _node_body,
        grid=(nb,),
        in_specs=[
            pl.BlockSpec((256, D), lambda i: (i, 0)),
            pl.BlockSpec((256, D), lambda i: (i, 0)),
            pl.BlockSpec((1, NS, 256, 16),
                         lambda i: (i // blk_per_half, 0, i % blk_per_half, 0)),
            pl.BlockSpec((256, 4), lambda i: (i, 0)),
            pl.BlockSpec((D, D), lambda i: (0, 0)),
            pl.BlockSpec((D, D), lambda i: (0, 0)),
            pl.BlockSpec((1, D), lambda i: (0, 0)),
            pl.BlockSpec((D, D), lambda i: (0, 0)),
            pl.BlockSpec((1, D), lambda i: (0, 0)),
        ],
        out_specs=[
            pl.BlockSpec((256, D), lambda i: (i, 0)),
            pl.BlockSpec((256, 4), lambda i: (i, 0)),
        ],
        out_shape=[
            jax.ShapeDtypeStruct((NP, D), jnp.float32),
            jax.ShapeDtypeStruct((NP, 4), jnp.float32),
        ],
        interpret=interpret,
    )(hinp, agg, cmparts, posp, wh1a, wh1b, bh1, wh2, bh2)


def _sc_gather_body(ad_hbm, as_hbm, posf_hbm, dst_hbm, src_hbm,
                    pre_hbm, rel_hbm,
                    pos_v, rows_a, rows_b, dsti_v, srci_v, rel_v,
                    sem_a, sem_b, sem_w):
    wid = lax.axis_index("s") * NC + lax.axis_index("c")
    base = wid * (EP // 32)
    pltpu.sync_copy(posf_hbm, pos_v)
    lanes = lax.iota(jnp.int32, L)
    z16 = jnp.zeros((L,), jnp.float32)

    def fire(g, slot):
        eb = base + g * GA

        @pl.when(g >= 2)
        def _():
            pltpu.make_async_copy(
                rows_a.at[slot], pre_hbm.at[pl.ds(base + (g - 2) * GA, GA)],
                sem_w).wait()

        pltpu.sync_copy(dst_hbm.at[pl.ds(eb, GA)], dsti_v.at[slot])
        pltpu.sync_copy(src_hbm.at[pl.ds(eb, GA)], srci_v.at[slot])
        pltpu.async_copy(ad_hbm.at[dsti_v.at[slot]], rows_a.at[slot], sem_a)
        pltpu.async_copy(as_hbm.at[srci_v.at[slot]], rows_b.at[slot], sem_b)

    fire(0, 0)

    def group(g, _):
        slot = lax.rem(g, 2)
        eb = base + g * GA

        @pl.when(g + 1 < NGA)
        def _():
            fire(g + 1, lax.rem(g + 1, 2))

        pltpu.make_async_copy(ad_hbm.at[dsti_v.at[slot]], rows_a.at[slot], sem_a).wait()
        pltpu.make_async_copy(as_hbm.at[srci_v.at[slot]], rows_b.at[slot], sem_b).wait()

        def sub(q, _):
            d16 = dsti_v[slot, pl.ds(q * L, L)]
            s16 = srci_v[slot, pl.ds(q * L, L)]
            d3 = d16 * 3
            s3 = s16 * 3
            rx = plsc.load_gather(pos_v, [d3]) - plsc.load_gather(pos_v, [s3])
            ry = plsc.load_gather(pos_v, [d3 + 1]) - plsc.load_gather(pos_v, [s3 + 1])
            rz = plsc.load_gather(pos_v, [d3 + 2]) - plsc.load_gather(pos_v, [s3 + 2])
            r4 = (q * L + lanes) * 4
            plsc.store_scatter(rel_v, [r4], rx)
            plsc.store_scatter(rel_v, [r4 + 1], ry)
            plsc.store_scatter(rel_v, [r4 + 2], rz)
            plsc.store_scatter(rel_v, [r4 + 3], z16)
            return 0

        lax.fori_loop(0, GA // L, sub, 0, unroll=True)

        def row(r, _):
            for k in range(D // L):
                sl = pl.ds(k * L, L)
                rows_a[slot, r, sl] = rows_a[slot, r, sl] + rows_b[slot, r, sl]
            return 0

        lax.fori_loop(0, GA, row, 0, unroll=4)
        pltpu.async_copy(rows_a.at[slot], pre_hbm.at[pl.ds(eb, GA)], sem_w)
        pltpu.sync_copy(rel_v, rel_hbm.at[pl.ds(eb * 4, GA * 4)])
        return 0

    lax.fori_loop(0, NGA, group, 0)
    for s in range(2):
        pltpu.make_async_copy(
            rows_a.at[s], pre_hbm.at[pl.ds(base + (NGA - 2 + s) * GA, GA)],
            sem_w).wait()


def _sc_gather_call(a_dst, a_src, posf, dstp, srcp):
    mesh = plsc.VectorSubcoreMesh(core_axis_name="c", subcore_axis_name="s")
    f = pl.kernel(
        _sc_gather_body,
        out_type=[
            jax.ShapeDtypeStruct((EP, D), jnp.float32),
            jax.ShapeDtypeStruct((EP * 4,), jnp.float32),
        ],
        mesh=mesh,
        scratch_types=[
            pltpu.VMEM((3 * NP,), jnp.float32),
            pltpu.VMEM((2, GA, D), jnp.float32),
            pltpu.VMEM((2, GA, D), jnp.float32),
            pltpu.VMEM((2, GA), jnp.int32),
            pltpu.VMEM((2, GA), jnp.int32),
            pltpu.VMEM((GA * 4,), jnp.float32),
            pltpu.SemaphoreType.DMA,
            pltpu.SemaphoreType.DMA,
            pltpu.SemaphoreType.DMA,
        ],
        compiler_params=pltpu.CompilerParams(needs_layout_passes=False),
    )
    return f(a_dst, a_src, posf, dstp, srcp)


def _sc_aggm2_body(mcat_hbm, dst_hbm, agg_hbm, tbl, stag, dsti_v):
    wid = lax.axis_index("s") * NC + lax.axis_index("c")
    z16 = jnp.zeros((L,), jnp.float32)

    def zero(k, _):
        for r in range(CS):
            tbl[r, pl.ds(k * L, L)] = z16
        return 0

    lax.fori_loop(0, NP // L, zero, 0)

    def group(g, _):
        eb = g * GE
        pltpu.sync_copy(dst_hbm.at[pl.ds(eb, GE)], dsti_v)
        pltpu.sync_copy(mcat_hbm.at[wid, :, pl.ds(eb, GE)], stag)

        def vec(q, _):
            d16 = dsti_v[pl.ds(q * L, L)]
            for c in range(CS):
                cc = jnp.full((L,), c, jnp.int32)
                val = stag[c, pl.ds(q * L, L)]
                plsc.addupdate_scatter(tbl, [cc, d16], val)
            return 0

        lax.fori_loop(0, GE // L, vec, 0, unroll=8)
        return 0

    lax.fori_loop(0, EP // GE, group, 0)
    pltpu.sync_copy(tbl, agg_hbm.at[wid])


def _sc_aggm2_call(mcat_t, dstp):
    mesh = plsc.VectorSubcoreMesh(core_axis_name="c", subcore_axis_name="s")
    f = pl.kernel(
        _sc_aggm2_body,
        out_type=[jax.ShapeDtypeStruct((32, CS, NP), jnp.float32)],
        mesh=mesh,
        scratch_types=[
            pltpu.VMEM((CS, NP), jnp.float32),
            pltpu.VMEM((CS, GE), jnp.float32),
            pltpu.VMEM((GE,), jnp.int32),
        ],
        compiler_params=pltpu.CompilerParams(needs_layout_passes=False),
    )
    return f(mcat_t, dstp)


def _sc_aggcm_body(mcat_hbm, dst_hbm, out_hbm, tbl, stag, dsti_v):
    h = lax.axis_index("c")
    sid = lax.axis_index("s")
    base_node = h * HALF
    lanes = lax.iota(jnp.int32, L)
    z16 = jnp.zeros((L,), jnp.float32)

    def zero(k, _):
        for r in range(16):
            tbl[r, pl.ds(k * L, L)] = z16
        return 0

    lax.fori_loop(0, HALF // L, zero, 0)

    def group(g, _):
        eb = sid * (EP // NS) + g * GE
        pltpu.sync_copy(dst_hbm.at[pl.ds(eb, GE)], dsti_v)
        pltpu.sync_copy(mcat_hbm.at[32, :, pl.ds(eb, GE)], stag.at[pl.ds(0, CS)])
        pltpu.sync_copy(mcat_hbm.at[33, :, pl.ds(eb, GE)], stag.at[pl.ds(CS, CS)])

        def vec(q, _):
            d16 = dsti_v[pl.ds(q * L, L)]
            loc = d16 - base_node
            owned = (loc >= 0) & (loc < HALF)
            locc = jnp.where(owned, loc, 0)
            for c in range(16):
                cc = jnp.full((L,), c, jnp.int32)
                val = stag[c, pl.ds(q * L, L)]
                plsc.addupdate_scatter(tbl, [cc, locc], val, mask=owned)
            return 0

        lax.fori_loop(0, GE // L, vec, 0)
        return 0

    lax.fori_loop(0, (EP // NS) // GE, group, 0)
    pltpu.sync_copy(tbl, out_hbm.at[h, sid])


def _sc_aggcm_call(mcat, dstp):
    mesh = plsc.VectorSubcoreMesh(core_axis_name="c", subcore_axis_name="s")
    f = pl.kernel(
        _sc_aggcm_body,
        out_type=[jax.ShapeDtypeStruct((NC, NS, 16, HALF), jnp.float32)],
        mesh=mesh,
        scratch_types=[
            pltpu.VMEM((16, HALF), jnp.float32),
            pltpu.VMEM((16, GE), jnp.float32),
            pltpu.VMEM((GE,), jnp.int32),
        ],
        compiler_params=pltpu.CompilerParams(needs_layout_passes=False),
    )
    return f(mcat, dstp)


def kernel(h, pos, edge_index, h_init, W_e1, b_e1, W_e2, b_e2,
           W_x1, b_x1, W_x2, b_x2, W_h1, b_h1, W_h2, b_h2):
    f32 = jnp.float32
    hp = jnp.pad(h, ((0, NP - N), (0, 0)))
    hip = jnp.pad(h_init, ((0, NP - N), (0, 0)))
    posf = jnp.pad(pos, ((0, NP - N), (0, 0))).reshape(3 * NP)
    src = edge_index[0]
    dst = edge_index[1]
    padi = jnp.full((EP - E,), PADNODE, jnp.int32)
    srcp = jnp.concatenate([src, padi])
    dstp = jnp.concatenate([dst, padi])

    w1a = W_e1[:D]
    w1b = W_e1[D:2 * D]
    wrow = W_e1[2 * D:2 * D + 1]
    be1 = b_e1.reshape(1, D)

    hinp, a_dst, a_src = _prep_call(hp, hip, w1a, w1b, be1)
    pre, relf = _sc_gather_call(a_dst, a_src, posf, dstp, srcp)
    (mcat,) = _edge_call(pre, relf.reshape(EP, 4), wrow, W_e2, b_e2.reshape(1, D),
                         W_x1, b_x1.reshape(1, D), W_x2.reshape(1, D),
                         b_x2.reshape(1, 1))
    mcat_t = jnp.transpose(mcat.reshape(EP, MW // CS, CS), (1, 2, 0))
    (agg_t,) = _sc_aggm2_call(mcat_t, dstp)
    agg = jnp.transpose(agg_t, (2, 0, 1)).reshape(NP, D)
    (cmparts_t,) = _sc_aggcm_call(mcat_t, dstp)
    cmparts = jnp.transpose(cmparts_t, (0, 1, 3, 2))
    posp4 = jnp.pad(pos, ((0, NP - N), (0, 1)))

    hout, pout = _node_call(hinp, agg, cmparts, posp4,
                            W_h1[:D], W_h1[D:], b_h1.reshape(1, D),
                            W_h2, b_h2.reshape(1, D))
    return (hout[:N], pout[:N, :3].astype(f32))


# speedup vs baseline: 1.0658x; 1.0658x over previous
import functools

import jax
import jax.numpy as jnp
from jax import lax
from jax.experimental import pallas as pl
from jax.experimental.pallas import tpu as pltpu
from jax.experimental.pallas import tpu_sc as plsc

N = 10000
E = 160000
D = 256

NC = 2
NS = 16
L = 16

NP = 10240
EP = 163840
PADNODE = 10200

HALF = NP // 2

MW = 272
CS = 8
GE = 1024

GA = 64
NGA = (EP // 32) // GA


def _silu(x):
    return x * jax.nn.sigmoid(x)


def _bdot(a, b):
    return jnp.dot(a.astype(jnp.bfloat16), b.astype(jnp.bfloat16),
                   preferred_element_type=jnp.float32)


def _prep_body(h_ref, hi_ref, w1a_ref, w1b_ref, be1_ref, hin_ref, ad_ref, as_ref):
    h_in = h_ref[...] + hi_ref[...]
    hin_ref[...] = h_in
    ad_ref[...] = _bdot(h_in, w1a_ref[...]) + be1_ref[...]
    as_ref[...] = _bdot(h_in, w1b_ref[...])


def _prep_call(hp, hip, w1a, w1b, be1, interpret=False):
    nb = NP // 256
    return pl.pallas_call(
        _prep_body,
        grid=(nb,),
        in_specs=[
            pl.BlockSpec((256, D), lambda i: (i, 0)),
            pl.BlockSpec((256, D), lambda i: (i, 0)),
            pl.BlockSpec((D, D), lambda i: (0, 0)),
            pl.BlockSpec((D, D), lambda i: (0, 0)),
            pl.BlockSpec((1, D), lambda i: (0, 0)),
        ],
        out_specs=[
            pl.BlockSpec((256, D), lambda i: (i, 0)),
            pl.BlockSpec((256, D), lambda i: (i, 0)),
            pl.BlockSpec((256, D), lambda i: (i, 0)),
        ],
        out_shape=[
            jax.ShapeDtypeStruct((NP, D), jnp.float32),
            jax.ShapeDtypeStruct((NP, D), jnp.float32),
            jax.ShapeDtypeStruct((NP, D), jnp.float32),
        ],
        interpret=interpret,
    )(hp, hip, w1a, w1b, be1)


def _edge_body(pre_ref, rel_ref, wrow_ref, we2_ref, be2_ref, wx1_ref, bx1_ref,
               wx2_ref, bx2_ref, mcat_ref):
    rel = rel_ref[...]
    d2 = jnp.sum(rel * rel, axis=1, keepdims=True)
    x = pre_ref[...] + d2 * wrow_ref[...]
    m1 = _silu(x)
    m2 = _silu(_bdot(m1, we2_ref[...]) + be2_ref[...])
    t = _silu(_bdot(m2, wx1_ref[...]) + bx1_ref[...])
    xw = jnp.sum(t * wx2_ref[...], axis=1, keepdims=True) + bx2_ref[...]
    nrows = rel.shape[0]
    mc = jnp.concatenate(
        [m2, rel * xw, jnp.ones((nrows, 1), jnp.float32),
         jnp.zeros((nrows, MW - D - 5), jnp.float32)], axis=1)
    mcat_ref[...] = jnp.transpose(mc).reshape(MW // CS, CS, nrows)


def _edge_call(pre, rel4, wrow, we2, be2, wx1, bx1, wx2r, bx2, interpret=False):
    eb = 1024
    nb = EP // eb
    return pl.pallas_call(
        _edge_body,
        grid=(nb,),
        in_specs=[
            pl.BlockSpec((eb, D), lambda i: (i, 0)),
            pl.BlockSpec((eb, 4), lambda i: (i, 0)),
            pl.BlockSpec((1, D), lambda i: (0, 0)),
            pl.BlockSpec((D, D), lambda i: (0, 0)),
            pl.BlockSpec((1, D), lambda i: (0, 0)),
            pl.BlockSpec((D, D), lambda i: (0, 0)),
            pl.BlockSpec((1, D), lambda i: (0, 0)),
            pl.BlockSpec((1, D), lambda i: (0, 0)),
            pl.BlockSpec((1, 1), lambda i: (0, 0)),
        ],
        out_specs=[
            pl.BlockSpec((MW // CS, CS, eb), lambda i: (0, 0, i)),
        ],
        out_shape=[
            jax.ShapeDtypeStruct((MW // CS, CS, EP), jnp.float32),
        ],
        interpret=interpret,
    )(pre, rel4, wrow, we2, be2, wx1, bx1, wx2r, bx2)


def _node_body(hin_ref, agg_ref, cmp_ref, pos_ref, wh1a_ref, wh1b_ref,
               bh1_ref, wh2_ref, bh2_ref, hout_ref, pout_ref):
    h_in = hin_ref[...]
    z = (_bdot(h_in, wh1a_ref[...])
         + _bdot(agg_ref[...], wh1b_ref[...])
         + bh1_ref[...])
    upd = _bdot(_silu(z), wh2_ref[...]) + bh2_ref[...]
    hout_ref[...] = h_in + upd
    cm = jnp.sum(cmp_ref[0], axis=0)
    cnt = jnp.maximum(cm[:, 4:5], 1.0)
    pout_ref[...] = pos_ref[...] + cm[:, 0:4] / cnt


def _node_call(hinp, agg, cmparts, posp, wh1a, wh1b, bh1, wh2, bh2, interpret=False):
    nb = NP // 256
    blk_per_half = HALF // 256
    return pl.pallas_call(
        _node_body,
        grid=(nb,),
        in_specs=[
            pl.BlockSpec((256, D), lambda i: (i, 0)),
            pl.BlockSpec((256, D), lambda i: (i, 0)),
            pl.BlockSpec((1, NS, 256, 16),
                         lambda i: (i // blk_per_half, 0, i % blk_per_half, 0)),
            pl.BlockSpec((256, 4), lambda i: (i, 0)),
            pl.BlockSpec((D, D), lambda i: (0, 0)),
            pl.BlockSpec((D, D), lambda i: (0, 0)),
            pl.BlockSpec((1, D), lambda i: (0, 0)),
            pl.BlockSpec((D, D), lambda i: (0, 0)),
            pl.BlockSpec((1, D), lambda i: (0, 0)),
        ],
        out_specs=[
            pl.BlockSpec((256, D), lambda i: (i, 0)),
            pl.BlockSpec((256, 4), lambda i: (i, 0)),
        ],
        out_shape=[
            jax.ShapeDtypeStruct((NP, D), jnp.float32),
            jax.ShapeDtypeStruct((NP, 4), jnp.float32),
        ],
        interpret=interpret,
    )(hinp, agg, cmparts, posp, wh1a, wh1b, bh1, wh2, bh2)


def _sc_gather_body(ad_hbm, as_hbm, posf_hbm, dst_hbm, src_hbm,
                    pre_hbm, rel_hbm,
                    pos_v, rows_a, rows_b, dsti_v, srci_v, rel_v,
                    sem_a, sem_b, sem_w):
    wid = lax.axis_index("s") * NC + lax.axis_index("c")
    base = wid * (EP // 32)
    pltpu.sync_copy(posf_hbm, pos_v)
    lanes = lax.iota(jnp.int32, L)
    z16 = jnp.zeros((L,), jnp.float32)

    def fire(g, slot):
        eb = base + g * GA

        @pl.when(g >= 2)
        def _():
            pltpu.make_async_copy(
                rows_a.at[slot], pre_hbm.at[pl.ds(base + (g - 2) * GA, GA)],
                sem_w).wait()

        pltpu.sync_copy(dst_hbm.at[pl.ds(eb, GA)], dsti_v.at[slot])
        pltpu.sync_copy(src_hbm.at[pl.ds(eb, GA)], srci_v.at[slot])
        pltpu.async_copy(ad_hbm.at[dsti_v.at[slot]], rows_a.at[slot], sem_a)
        pltpu.async_copy(as_hbm.at[srci_v.at[slot]], rows_b.at[slot], sem_b)

    fire(0, 0)

    def group(g, _):
        slot = lax.rem(g, 2)
        eb = base + g * GA

        @pl.when(g + 1 < NGA)
        def _():
            fire(g + 1, lax.rem(g + 1, 2))

        pltpu.make_async_copy(ad_hbm.at[dsti_v.at[slot]], rows_a.at[slot], sem_a).wait()
        pltpu.make_async_copy(as_hbm.at[srci_v.at[slot]], rows_b.at[slot], sem_b).wait()

        def sub(q, _):
            d16 = dsti_v[slot, pl.ds(q * L, L)]
            s16 = srci_v[slot, pl.ds(q * L, L)]
            d3 = d16 * 3
            s3 = s16 * 3
            rx = plsc.load_gather(pos_v, [d3]) - plsc.load_gather(pos_v, [s3])
            ry = plsc.load_gather(pos_v, [d3 + 1]) - plsc.load_gather(pos_v, [s3 + 1])
            rz = plsc.load_gather(pos_v, [d3 + 2]) - plsc.load_gather(pos_v, [s3 + 2])
            r4 = (q * L + lanes) * 4
            plsc.store_scatter(rel_v, [r4], rx)
            plsc.store_scatter(rel_v, [r4 + 1], ry)
            plsc.store_scatter(rel_v, [r4 + 2], rz)
            plsc.store_scatter(rel_v, [r4 + 3], z16)
            return 0

        lax.fori_loop(0, GA // L, sub, 0, unroll=True)

        def row(r, _):
            for k in range(D // L):
                sl = pl.ds(k * L, L)
                rows_a[slot, r, sl] = rows_a[slot, r, sl] + rows_b[slot, r, sl]
            return 0

        lax.fori_loop(0, GA, row, 0, unroll=4)
        pltpu.async_copy(rows_a.at[slot], pre_hbm.at[pl.ds(eb, GA)], sem_w)
        pltpu.sync_copy(rel_v, rel_hbm.at[pl.ds(eb * 4, GA * 4)])
        return 0

    lax.fori_loop(0, NGA, group, 0)
    for s in range(2):
        pltpu.make_async_copy(
            rows_a.at[s], pre_hbm.at[pl.ds(base + (NGA - 2 + s) * GA, GA)],
            sem_w).wait()


def _sc_gather_call(a_dst, a_src, posf, dstp, srcp):
    mesh = plsc.VectorSubcoreMesh(core_axis_name="c", subcore_axis_name="s")
    f = pl.kernel(
        _sc_gather_body,
        out_type=[
            jax.ShapeDtypeStruct((EP, D), jnp.float32),
            jax.ShapeDtypeStruct((EP * 4,), jnp.float32),
        ],
        mesh=mesh,
        scratch_types=[
            pltpu.VMEM((3 * NP,), jnp.float32),
            pltpu.VMEM((2, GA, D), jnp.float32),
            pltpu.VMEM((2, GA, D), jnp.float32),
            pltpu.VMEM((2, GA), jnp.int32),
            pltpu.VMEM((2, GA), jnp.int32),
            pltpu.VMEM((GA * 4,), jnp.float32),
            pltpu.SemaphoreType.DMA,
            pltpu.SemaphoreType.DMA,
            pltpu.SemaphoreType.DMA,
        ],
        compiler_params=pltpu.CompilerParams(needs_layout_passes=False),
    )
    return f(a_dst, a_src, posf, dstp, srcp)


def _sc_aggm2_body(mcat_hbm, dst_hbm, agg_hbm, tbl, stag, dsti_v):
    wid = lax.axis_index("s") * NC + lax.axis_index("c")
    z16 = jnp.zeros((L,), jnp.float32)

    def zero(k, _):
        for r in range(CS):
            tbl[r, pl.ds(k * L, L)] = z16
        return 0

    lax.fori_loop(0, NP // L, zero, 0)

    def group(g, _):
        eb = g * GE
        pltpu.sync_copy(dst_hbm.at[pl.ds(eb, GE)], dsti_v)
        pltpu.sync_copy(mcat_hbm.at[wid, :, pl.ds(eb, GE)], stag)

        def vec(q, _):
            d16 = dsti_v[pl.ds(q * L, L)]
            for c in range(CS):
                cc = jnp.full((L,), c, jnp.int32)
                val = stag[c, pl.ds(q * L, L)]
                plsc.addupdate_scatter(tbl, [cc, d16], val)
            return 0

        lax.fori_loop(0, GE // L, vec, 0, unroll=8)
        return 0

    lax.fori_loop(0, EP // GE, group, 0)
    pltpu.sync_copy(tbl, agg_hbm.at[wid])


def _sc_aggm2_call(mcat_t, dstp):
    mesh = plsc.VectorSubcoreMesh(core_axis_name="c", subcore_axis_name="s")
    f = pl.kernel(
        _sc_aggm2_body,
        out_type=[jax.ShapeDtypeStruct((32, CS, NP), jnp.float32)],
        mesh=mesh,
        scratch_types=[
            pltpu.VMEM((CS, NP), jnp.float32),
            pltpu.VMEM((CS, GE), jnp.float32),
            pltpu.VMEM((GE,), jnp.int32),
        ],
        compiler_params=pltpu.CompilerParams(needs_layout_passes=False),
    )
    return f(mcat_t, dstp)


def _sc_aggcm_body(mcat_hbm, dst_hbm, out_hbm, tbl, stag, dsti_v):
    h = lax.axis_index("c")
    sid = lax.axis_index("s")
    base_node = h * HALF
    lanes = lax.iota(jnp.int32, L)
    z16 = jnp.zeros((L,), jnp.float32)

    def zero(k, _):
        for r in range(16):
            tbl[r, pl.ds(k * L, L)] = z16
        return 0

    lax.fori_loop(0, HALF // L, zero, 0)

    def group(g, _):
        eb = sid * (EP // NS) + g * GE
        pltpu.sync_copy(dst_hbm.at[pl.ds(eb, GE)], dsti_v)
        pltpu.sync_copy(mcat_hbm.at[32, :, pl.ds(eb, GE)], stag.at[pl.ds(0, CS)])
        pltpu.sync_copy(mcat_hbm.at[33, :, pl.ds(eb, GE)], stag.at[pl.ds(CS, CS)])

        def vec(q, _):
            d16 = dsti_v[pl.ds(q * L, L)]
            loc = d16 - base_node
            owned = (loc >= 0) & (loc < HALF)
            locc = jnp.where(owned, loc, 0)
            for c in range(16):
                cc = jnp.full((L,), c, jnp.int32)
                val = stag[c, pl.ds(q * L, L)]
                plsc.addupdate_scatter(tbl, [cc, locc], val, mask=owned)
            return 0

        lax.fori_loop(0, GE // L, vec, 0)
        return 0

    lax.fori_loop(0, (EP // NS) // GE, group, 0)
    pltpu.sync_copy(tbl, out_hbm.at[h, sid])


def _sc_aggcm_call(mcat, dstp):
    mesh = plsc.VectorSubcoreMesh(core_axis_name="c", subcore_axis_name="s")
    f = pl.kernel(
        _sc_aggcm_body,
        out_type=[jax.ShapeDtypeStruct((NC, NS, 16, HALF), jnp.float32)],
        mesh=mesh,
        scratch_types=[
            pltpu.VMEM((16, HALF), jnp.float32),
            pltpu.VMEM((16, GE), jnp.float32),
            pltpu.VMEM((GE,), jnp.int32),
        ],
        compiler_params=pltpu.CompilerParams(needs_layout_passes=False),
    )
    return f(mcat, dstp)


def kernel(h, pos, edge_index, h_init, W_e1, b_e1, W_e2, b_e2,
           W_x1, b_x1, W_x2, b_x2, W_h1, b_h1, W_h2, b_h2):
    f32 = jnp.float32
    hp = jnp.pad(h, ((0, NP - N), (0, 0)))
    hip = jnp.pad(h_init, ((0, NP - N), (0, 0)))
    posf = jnp.pad(pos, ((0, NP - N), (0, 0))).reshape(3 * NP)
    src = edge_index[0]
    dst = edge_index[1]
    padi = jnp.full((EP - E,), PADNODE, jnp.int32)
    srcp = jnp.concatenate([src, padi])
    dstp = jnp.concatenate([dst, padi])

    w1a = W_e1[:D]
    w1b = W_e1[D:2 * D]
    wrow = W_e1[2 * D:2 * D + 1]
    be1 = b_e1.reshape(1, D)

    hinp, a_dst, a_src = _prep_call(hp, hip, w1a, w1b, be1)
    pre, relf = _sc_gather_call(a_dst, a_src, posf, dstp, srcp)
    (mcat,) = _edge_call(pre, relf.reshape(EP, 4), wrow, W_e2, b_e2.reshape(1, D),
                         W_x1, b_x1.reshape(1, D), W_x2.reshape(1, D),
                         b_x2.reshape(1, 1))
    mcat_t = mcat
    (agg_t,) = _sc_aggm2_call(mcat_t, dstp)
    agg = jnp.transpose(agg_t, (2, 0, 1)).reshape(NP, D)
    (cmparts_t,) = _sc_aggcm_call(mcat_t, dstp)
    cmparts = jnp.transpose(cmparts_t, (0, 1, 3, 2))
    posp4 = jnp.pad(pos, ((0, NP - N), (0, 1)))

    hout, pout = _node_call(hinp, agg, cmparts, posp4,
                            W_h1[:D], W_h1[D:], b_h1.reshape(1, D),
                            W_h2, b_h2.reshape(1, D))
    return (hout[:N], pout[:N, :3].astype(f32))
